# Initial kernel scaffold; baseline (speedup 1.0000x reference)
import functools

import jax
import jax.numpy as jnp
from jax.experimental import pallas as pl
from jax.experimental.pallas import tpu as pltpu

NUM_E = 8
TOPK = 2
NGRP = 4
EPG = NUM_E // NGRP
T = 2048
H = 2048
FFN = 1024
SFFN = 2048


def _routing_body(hid_ref, gate_ref, wm_ref):
    x = hid_ref[...]
    gw = gate_ref[...]
    logits = jax.lax.dot_general(
        x, gw, (((1,), (1,)), ((), ())), preferred_element_type=jnp.float32)
    m = jnp.max(logits, axis=1, keepdims=True)
    ex = jnp.exp(logits - m)
    scores = ex / jnp.sum(ex, axis=1, keepdims=True)

    r = jax.lax.broadcasted_iota(jnp.int32, (NUM_E, NUM_E), 0)
    c = jax.lax.broadcasted_iota(jnp.int32, (NUM_E, NUM_E), 1)
    swap = ((r ^ 1) == c).astype(jnp.float32)
    swapped = jax.lax.dot_general(
        scores, swap, (((1,), (0,)), ((), ())), preferred_element_type=jnp.float32)
    gs = jnp.maximum(scores, swapped)

    lane = jax.lax.broadcasted_iota(jnp.int32, scores.shape, 1)
    gidx = lane // EPG
    big = jnp.int32(1 << 20)
    neg = jnp.float32(-jnp.inf)

    v1 = jnp.max(gs, axis=1, keepdims=True)
    g1 = jnp.min(jnp.where(gs == v1, gidx, big), axis=1, keepdims=True)
    gs2 = jnp.where(gidx == g1, neg, gs)
    v2 = jnp.max(gs2, axis=1, keepdims=True)
    g2 = jnp.min(jnp.where(gs2 == v2, gidx, big), axis=1, keepdims=True)
    gmask = (gidx == g1) | (gidx == g2)

    ms = jnp.where(gmask, scores, 0.0)
    w1 = jnp.max(ms, axis=1, keepdims=True)
    e1 = jnp.min(jnp.where(ms == w1, lane, big), axis=1, keepdims=True)
    ms2 = jnp.where(lane == e1, -1.0, ms)
    w2 = jnp.max(ms2, axis=1, keepdims=True)
    e2 = jnp.min(jnp.where(ms2 == w2, lane, big), axis=1, keepdims=True)

    wm = jnp.where(lane == e1, w1, 0.0) + jnp.where(lane == e2, w2, 0.0)
    wm_ref[...] = wm


def _routing(hidden, gate_weight):
    return pl.pallas_call(
        _routing_body,
        out_shape=jax.ShapeDtypeStruct((T, NUM_E), jnp.float32),
    )(hidden, gate_weight)


def _dense_expert_body(hid_ref, wg_ref, wu_ref, wd_ref, wm_ref, out_ref):
    e = pl.program_id(0)
    c = pl.program_id(1)

    @pl.when((e == 0) & (c == 0))
    def _():
        out_ref[...] = jnp.zeros_like(out_ref)

    x = hid_ref[...]
    g = jax.lax.dot_general(
        x, wg_ref[0], (((1,), (1,)), ((), ())), preferred_element_type=jnp.float32)
    u = jax.lax.dot_general(
        x, wu_ref[0], (((1,), (1,)), ((), ())), preferred_element_type=jnp.float32)
    a = g * jax.nn.sigmoid(g) * u
    lane = jax.lax.broadcasted_iota(jnp.int32, (T, NUM_E), 1)
    w = jnp.sum(jnp.where(lane == e, wm_ref[...], 0.0), axis=1, keepdims=True)
    a = a * w
    out_ref[...] += jax.lax.dot_general(
        a, wd_ref[0], (((1,), (1,)), ((), ())), preferred_element_type=jnp.float32)


def _dense_experts(hidden, Wg, Wu, Wd, wm):
    fc = 256
    nc = FFN // fc
    return pl.pallas_call(
        _dense_expert_body,
        grid=(NUM_E, nc),
        in_specs=[
            pl.BlockSpec((T, H), lambda e, c: (0, 0)),
            pl.BlockSpec((1, fc, H), lambda e, c: (e, c, 0)),
            pl.BlockSpec((1, fc, H), lambda e, c: (e, c, 0)),
            pl.BlockSpec((1, H, fc), lambda e, c: (e, 0, c)),
            pl.BlockSpec((T, NUM_E), lambda e, c: (0, 0)),
        ],
        out_specs=pl.BlockSpec((T, H), lambda e, c: (0, 0)),
        out_shape=jax.ShapeDtypeStruct((T, H), jnp.float32),
    )(hidden, Wg, Wu, Wd, wm)


def _shared_body(hid_ref, sg_ref, su_ref, sd_ref, out_ref):
    c = pl.program_id(0)

    @pl.when(c == 0)
    def _():
        out_ref[...] = jnp.zeros_like(out_ref)

    x = hid_ref[...]
    g = jax.lax.dot_general(
        x, sg_ref[...], (((1,), (1,)), ((), ())), preferred_element_type=jnp.float32)
    u = jax.lax.dot_general(
        x, su_ref[...], (((1,), (1,)), ((), ())), preferred_element_type=jnp.float32)
    a = g * jax.nn.sigmoid(g) * u
    out_ref[...] += jax.lax.dot_general(
        a, sd_ref[...], (((1,), (1,)), ((), ())), preferred_element_type=jnp.float32)


def _shared_expert(hidden, Sg, Su, Sd):
    fc = 256
    nc = SFFN // fc
    return pl.pallas_call(
        _shared_body,
        grid=(nc,),
        in_specs=[
            pl.BlockSpec((T, H), lambda c: (0, 0)),
            pl.BlockSpec((fc, H), lambda c: (c, 0)),
            pl.BlockSpec((fc, H), lambda c: (c, 0)),
            pl.BlockSpec((H, fc), lambda c: (0, c)),
        ],
        out_specs=pl.BlockSpec((T, H), lambda c: (0, 0)),
        out_shape=jax.ShapeDtypeStruct((T, H), jnp.float32),
    )(hidden, Sg, Su, Sd)


def _add_body(a_ref, b_ref, out_ref):
    out_ref[...] = a_ref[...] + b_ref[...]


def _add(a, b):
    bt = 256
    return pl.pallas_call(
        _add_body,
        grid=(T // bt,),
        in_specs=[
            pl.BlockSpec((bt, H), lambda i: (i, 0)),
            pl.BlockSpec((bt, H), lambda i: (i, 0)),
        ],
        out_specs=pl.BlockSpec((bt, H), lambda i: (i, 0)),
        out_shape=jax.ShapeDtypeStruct((T, H), jnp.float32),
    )(a, b)


def kernel(hidden_states, gate_weight, Wg, Wu, Wd, Sg, Su, Sd):
    b, s, h = hidden_states.shape
    hidden = hidden_states.reshape(-1, h)
    wm = _routing(hidden, gate_weight)
    routed = _dense_experts(hidden, Wg, Wu, Wd, wm)
    shared = _shared_expert(hidden, Sg, Su, Sd)
    return _add(routed, shared).reshape(b, s, h)


# speedup vs baseline: 1.7039x; 1.7039x over previous
import functools

import jax
import jax.numpy as jnp
from jax.experimental import pallas as pl
from jax.experimental.pallas import tpu as pltpu

NUM_E = 8
TOPK = 2
NGRP = 4
EPG = NUM_E // NGRP
T = 2048
H = 2048
FFN = 1024
SFFN = 2048


def _routing_body(hid_ref, gate_ref, wm_ref):
    x = hid_ref[...]
    gw = gate_ref[...]
    logits = jax.lax.dot_general(
        x, gw, (((1,), (1,)), ((), ())),
        preferred_element_type=jnp.float32,
        precision=jax.lax.Precision.HIGHEST)
    m = jnp.max(logits, axis=1, keepdims=True)
    ex = jnp.exp(logits - m)
    scores = ex / jnp.sum(ex, axis=1, keepdims=True)

    r = jax.lax.broadcasted_iota(jnp.int32, (NUM_E, NUM_E), 0)
    c = jax.lax.broadcasted_iota(jnp.int32, (NUM_E, NUM_E), 1)
    swap = ((r ^ 1) == c).astype(jnp.float32)
    swapped = jax.lax.dot_general(
        scores, swap, (((1,), (0,)), ((), ())), preferred_element_type=jnp.float32)
    gs = jnp.maximum(scores, swapped)

    lane = jax.lax.broadcasted_iota(jnp.int32, scores.shape, 1)
    gidx = lane // EPG
    big = jnp.int32(1 << 20)
    neg = jnp.float32(-jnp.inf)

    v1 = jnp.max(gs, axis=1, keepdims=True)
    g1 = jnp.min(jnp.where(gs == v1, gidx, big), axis=1, keepdims=True)
    gs2 = jnp.where(gidx == g1, neg, gs)
    v2 = jnp.max(gs2, axis=1, keepdims=True)
    g2 = jnp.min(jnp.where(gs2 == v2, gidx, big), axis=1, keepdims=True)
    gmask = (gidx == g1) | (gidx == g2)

    ms = jnp.where(gmask, scores, 0.0)
    w1 = jnp.max(ms, axis=1, keepdims=True)
    e1 = jnp.min(jnp.where(ms == w1, lane, big), axis=1, keepdims=True)
    ms2 = jnp.where(lane == e1, -1.0, ms)
    w2 = jnp.max(ms2, axis=1, keepdims=True)
    e2 = jnp.min(jnp.where(ms2 == w2, lane, big), axis=1, keepdims=True)

    wm = jnp.where(lane == e1, w1, 0.0) + jnp.where(lane == e2, w2, 0.0)
    wm_ref[...] = wm


def _routing(hidden, gate_weight):
    return pl.pallas_call(
        _routing_body,
        out_shape=jax.ShapeDtypeStruct((T, NUM_E), jnp.float32),
    )(hidden, gate_weight)


def _dense_expert_body(hid_ref, wg_ref, wu_ref, wd_ref, wm_ref, out_ref):
    e = pl.program_id(0)
    c = pl.program_id(1)

    @pl.when((e == 0) & (c == 0))
    def _():
        out_ref[...] = jnp.zeros_like(out_ref)

    x = hid_ref[...]
    g = jax.lax.dot_general(
        x, wg_ref[0], (((1,), (1,)), ((), ())), preferred_element_type=jnp.float32)
    u = jax.lax.dot_general(
        x, wu_ref[0], (((1,), (1,)), ((), ())), preferred_element_type=jnp.float32)
    a = g * jax.nn.sigmoid(g) * u
    lane = jax.lax.broadcasted_iota(jnp.int32, (T, NUM_E), 1)
    w = jnp.sum(jnp.where(lane == e, wm_ref[...], 0.0), axis=1, keepdims=True)
    a = a * w
    out_ref[...] += jax.lax.dot_general(
        a, wd_ref[0], (((1,), (1,)), ((), ())), preferred_element_type=jnp.float32)


def _dense_experts(hidden, Wg, Wu, Wd, wm):
    fc = 256
    nc = FFN // fc
    return pl.pallas_call(
        _dense_expert_body,
        grid=(NUM_E, nc),
        in_specs=[
            pl.BlockSpec((T, H), lambda e, c: (0, 0)),
            pl.BlockSpec((1, fc, H), lambda e, c: (e, c, 0)),
            pl.BlockSpec((1, fc, H), lambda e, c: (e, c, 0)),
            pl.BlockSpec((1, H, fc), lambda e, c: (e, 0, c)),
            pl.BlockSpec((T, NUM_E), lambda e, c: (0, 0)),
        ],
        out_specs=pl.BlockSpec((T, H), lambda e, c: (0, 0)),
        out_shape=jax.ShapeDtypeStruct((T, H), jnp.float32),
    )(hidden, Wg, Wu, Wd, wm)


def _shared_body(hid_ref, sg_ref, su_ref, sd_ref, out_ref):
    c = pl.program_id(0)

    @pl.when(c == 0)
    def _():
        out_ref[...] = jnp.zeros_like(out_ref)

    x = hid_ref[...]
    g = jax.lax.dot_general(
        x, sg_ref[...], (((1,), (1,)), ((), ())), preferred_element_type=jnp.float32)
    u = jax.lax.dot_general(
        x, su_ref[...], (((1,), (1,)), ((), ())), preferred_element_type=jnp.float32)
    a = g * jax.nn.sigmoid(g) * u
    out_ref[...] += jax.lax.dot_general(
        a, sd_ref[...], (((1,), (1,)), ((), ())), preferred_element_type=jnp.float32)


def _shared_expert(hidden, Sg, Su, Sd):
    fc = 256
    nc = SFFN // fc
    return pl.pallas_call(
        _shared_body,
        grid=(nc,),
        in_specs=[
            pl.BlockSpec((T, H), lambda c: (0, 0)),
            pl.BlockSpec((fc, H), lambda c: (c, 0)),
            pl.BlockSpec((fc, H), lambda c: (c, 0)),
            pl.BlockSpec((H, fc), lambda c: (0, c)),
        ],
        out_specs=pl.BlockSpec((T, H), lambda c: (0, 0)),
        out_shape=jax.ShapeDtypeStruct((T, H), jnp.float32),
    )(hidden, Sg, Su, Sd)


def _add_body(a_ref, b_ref, out_ref):
    out_ref[...] = a_ref[...] + b_ref[...]


def _add(a, b):
    bt = 256
    return pl.pallas_call(
        _add_body,
        grid=(T // bt,),
        in_specs=[
            pl.BlockSpec((bt, H), lambda i: (i, 0)),
            pl.BlockSpec((bt, H), lambda i: (i, 0)),
        ],
        out_specs=pl.BlockSpec((bt, H), lambda i: (i, 0)),
        out_shape=jax.ShapeDtypeStruct((T, H), jnp.float32),
    )(a, b)


def kernel(hidden_states, gate_weight, Wg, Wu, Wd, Sg, Su, Sd):
    b, s, h = hidden_states.shape
    hidden = hidden_states.reshape(-1, h)
    wm = _routing(hidden, gate_weight)
    routed = _dense_experts(hidden, Wg, Wu, Wd, wm)
    shared = _shared_expert(hidden, Sg, Su, Sd)
    return _add(routed, shared).reshape(b, s, h)
